# Initial kernel scaffold; baseline (speedup 1.0000x reference)
import jax
import jax.numpy as jnp
from jax.experimental import pallas as pl


def kernel(x_embed, maben, prompt_key, ln1_g, ln1_b, Wqkv, Wo, Wo_b, ln2_g, ln2_b, W1, b1, W2, b2, pre_out_W, pre_out_b, key_W, key_b, query_W, query_b):
    raise NotImplementedError("write your pallas kernel here")


# speedup vs baseline: 2.5285x; 2.5285x over previous
import math

import jax
import jax.numpy as jnp
from jax.experimental import pallas as pl

D = 768
DEPTH = 3
HEADS = 12
DH = 64
NUM_P2 = 32
MABEN_N = 256
TOP_K = 5
S_REAL = 453
S_PAD = 456
NEG = -1e30


def _gelu(x):
    return x * 0.5 * (1.0 + jax.lax.erf(x * (1.0 / math.sqrt(2.0))))


def _ln(x, g, b, eps=1e-5):
    m = jnp.mean(x, axis=-1, keepdims=True)
    v = jnp.mean((x - m) ** 2, axis=-1, keepdims=True)
    return (x - m) * jax.lax.rsqrt(v + eps) * g + b


def _layer_body(x_ref, g1_ref, b1_ref, wqkv_ref, wo_ref, wob_ref, g2_ref,
                b2_ref, w1_ref, bm1_ref, w2_ref, bm2_ref, o_ref):
    x = x_ref[...]
    h = _ln(x, g1_ref[...], b1_ref[...]).astype(jnp.bfloat16)
    qkv = jax.lax.dot(h, wqkv_ref[...],
                      preferred_element_type=jnp.float32).astype(jnp.bfloat16)
    col = jax.lax.broadcasted_iota(jnp.int32, (S_PAD, S_PAD), 1)
    mask = jnp.where(col < S_REAL, 0.0, NEG)
    scale = 1.0 / math.sqrt(DH)
    outs = []
    for hd in range(HEADS):
        q = qkv[:, hd * DH:(hd + 1) * DH]
        k = qkv[:, D + hd * DH:D + (hd + 1) * DH]
        v = qkv[:, 2 * D + hd * DH:2 * D + (hd + 1) * DH]
        dots = jax.lax.dot_general(
            q, k, (((1,), (1,)), ((), ())),
            preferred_element_type=jnp.float32) * scale + mask
        a = jax.nn.softmax(dots, axis=-1).astype(jnp.bfloat16)
        outs.append(jax.lax.dot(a, v, preferred_element_type=jnp.float32))
    o = jnp.concatenate(outs, axis=1).astype(jnp.bfloat16)
    x = jax.lax.dot(o, wo_ref[...],
                    preferred_element_type=jnp.float32) + wob_ref[...] + x
    h2 = _ln(x, g2_ref[...], b2_ref[...]).astype(jnp.bfloat16)
    t = jax.lax.dot(h2, w1_ref[...],
                    preferred_element_type=jnp.float32) + bm1_ref[...]
    t = _gelu(t).astype(jnp.bfloat16)
    o_ref[...] = jax.lax.dot(t, w2_ref[...],
                             preferred_element_type=jnp.float32) + bm2_ref[...] + x


def _cls_body(x_ref, g1_ref, b1_ref, wq_ref, wkv_ref, wo_ref, wob_ref, g2_ref,
              b2_ref, w1_ref, bm1_ref, w2_ref, bm2_ref, o_ref):
    x = x_ref[...]
    h = _ln(x, g1_ref[...], b1_ref[...]).astype(jnp.bfloat16)
    kv = jax.lax.dot(h, wkv_ref[...],
                     preferred_element_type=jnp.float32).astype(jnp.bfloat16)
    qc = jax.lax.dot(h[0:1, :], wq_ref[...],
                     preferred_element_type=jnp.float32).astype(jnp.bfloat16)
    col = jax.lax.broadcasted_iota(jnp.int32, (1, S_PAD), 1)
    mask = jnp.where(col < S_REAL, 0.0, NEG)
    scale = 1.0 / math.sqrt(DH)
    outs = []
    for hd in range(HEADS):
        q = qc[:, hd * DH:(hd + 1) * DH]
        k = kv[:, hd * DH:(hd + 1) * DH]
        v = kv[:, D + hd * DH:D + (hd + 1) * DH]
        dots = jax.lax.dot_general(
            q, k, (((1,), (1,)), ((), ())),
            preferred_element_type=jnp.float32) * scale + mask
        a = jax.nn.softmax(dots, axis=-1).astype(jnp.bfloat16)
        outs.append(jax.lax.dot(a, v, preferred_element_type=jnp.float32))
    o = jnp.concatenate(outs, axis=1).astype(jnp.bfloat16)
    xc = x[0:1, :]
    xc = jax.lax.dot(o, wo_ref[...],
                     preferred_element_type=jnp.float32) + wob_ref[...] + xc
    h2 = _ln(xc, g2_ref[...], b2_ref[...]).astype(jnp.bfloat16)
    t = jax.lax.dot(h2, w1_ref[...],
                    preferred_element_type=jnp.float32) + bm1_ref[...]
    t = _gelu(t).astype(jnp.bfloat16)
    o_ref[...] = jax.lax.dot(t, w2_ref[...],
                             preferred_element_type=jnp.float32) + bm2_ref[...] + xc


def _xm_sim_body(x_ref, pk_ref, xm_ref, sim_ref):
    xm = jnp.mean(x_ref[...], axis=1)
    xm_ref[...] = xm
    xn = xm * jax.lax.rsqrt(
        jnp.maximum(jnp.sum(xm * xm, axis=-1, keepdims=True), 1e-12))
    pk = pk_ref[...]
    pn = pk * jax.lax.rsqrt(
        jnp.maximum(jnp.sum(pk * pk, axis=-1, keepdims=True), 1e-12))
    sim_ref[...] = jax.lax.dot_general(
        xn.astype(jnp.bfloat16), pn.astype(jnp.bfloat16),
        (((1,), (1,)), ((), ())),
        preferred_element_type=jnp.float32)


def _head_body(cls_ref, xm_ref, pow_ref, pob_ref, maben_ref, kw_ref, kb_ref,
               qw_ref, qb_ref, pe_ref):
    cls = cls_ref[...].astype(jnp.bfloat16)
    corr = jax.lax.dot(cls, pow_ref[...],
                       preferred_element_type=jnp.float32) + pob_ref[...]
    xm = xm_ref[...]
    k_ = jax.lax.dot(xm.astype(jnp.bfloat16), kw_ref[...],
                     preferred_element_type=jnp.float32) + kb_ref[...]
    kq = jax.lax.dot_general(
        k_.astype(jnp.bfloat16), qw_ref[...], (((1,), (1,)), ((), ())),
        preferred_element_type=jnp.float32)
    qbk = jnp.sum(k_ * qb_ref[...], axis=1, keepdims=True)
    inv = 1.0 / math.sqrt(D)
    for o in range(NUM_P2):
        c = corr[:, o * MABEN_N:(o + 1) * MABEN_N]
        m = jax.nn.softmax(c, axis=-1).astype(jnp.bfloat16)
        pg = jax.lax.dot(m, maben_ref[...], preferred_element_type=jnp.float32)
        s = (jnp.sum(pg * kq, axis=1, keepdims=True) + qbk) * inv
        pe_ref[:, o, :] = pg * jax.nn.sigmoid(s)


def _topk_body(sim_ref, pe_ref, out_ref):
    sim = sim_ref[...]
    b = sim.shape[0]
    colio = jax.lax.broadcasted_iota(jnp.int32, (b, NUM_P2), 1)
    pe = pe_ref[...]
    masked = sim
    for k in range(TOP_K):
        mx = jnp.max(masked, axis=1, keepdims=True)
        eq = masked >= mx
        mn = jnp.min(jnp.where(eq, colio, NUM_P2), axis=1, keepdims=True)
        pick = colio == mn
        w = pick.astype(jnp.float32)
        out_ref[:, k, :] = jnp.sum(w[:, :, None] * pe, axis=1)
        masked = jnp.where(pick, NEG, masked)


def _full2d(a):
    return pl.BlockSpec(a.shape, lambda i: (0,) * a.ndim)


def _layer_call(h, body, wargs):
    bsz = h.shape[0]
    return pl.pallas_call(
        body,
        grid=(bsz,),
        in_specs=[pl.BlockSpec((None, S_PAD, D), lambda i: (i, 0, 0))] +
                 [_full2d(w) for w in wargs],
        out_specs=pl.BlockSpec((None, S_PAD, D), lambda i: (i, 0, 0)),
        out_shape=jax.ShapeDtypeStruct((bsz, S_PAD, D), jnp.float32),
    )(h, *wargs)


def _cls_call(h, wargs):
    bsz = h.shape[0]
    return pl.pallas_call(
        _cls_body,
        grid=(bsz,),
        in_specs=[pl.BlockSpec((None, S_PAD, D), lambda i: (i, 0, 0))] +
                 [_full2d(w) for w in wargs],
        out_specs=pl.BlockSpec((None, 1, D), lambda i: (i, 0, 0)),
        out_shape=jax.ShapeDtypeStruct((bsz, 1, D), jnp.float32),
    )(h, *wargs).reshape(bsz, D)


def kernel(x_embed, maben, prompt_key, ln1_g, ln1_b, Wqkv, Wo, Wo_b, ln2_g,
           ln2_b, W1, b1, W2, b2, pre_out_W, pre_out_b, key_W, key_b, query_W,
           query_b):
    f32 = jnp.float32
    bf = jnp.bfloat16
    bsz = x_embed.shape[0]

    pt = jnp.broadcast_to(maben[None], (bsz, MABEN_N, D))
    pad = jnp.zeros((bsz, S_PAD - S_REAL, D), f32)
    h = jnp.concatenate([x_embed, pt, pad], axis=1)

    def layer_weights(l):
        return (ln1_g[l].reshape(1, D), ln1_b[l].reshape(1, D),
                Wqkv[l].astype(bf), Wo[l].astype(bf), Wo_b[l].reshape(1, D),
                ln2_g[l].reshape(1, D), ln2_b[l].reshape(1, D),
                W1[l].astype(bf), b1[l].reshape(1, D),
                W2[l].astype(bf), b2[l].reshape(1, D))

    for l in range(DEPTH - 1):
        h = _layer_call(h, _layer_body, layer_weights(l))

    lw = layer_weights(DEPTH - 1)
    wq = Wqkv[DEPTH - 1][:, :D].astype(bf)
    wkv = Wqkv[DEPTH - 1][:, D:].astype(bf)
    cls = _cls_call(h, (lw[0], lw[1], wq, wkv) + lw[3:])

    xm, sim = pl.pallas_call(
        _xm_sim_body,
        out_shape=(jax.ShapeDtypeStruct((bsz, D), f32),
                   jax.ShapeDtypeStruct((bsz, NUM_P2), f32)),
    )(x_embed, prompt_key)

    pe = pl.pallas_call(
        _head_body,
        out_shape=jax.ShapeDtypeStruct((bsz, NUM_P2, D), f32),
    )(cls, xm, pre_out_W.astype(bf), pre_out_b.reshape(1, NUM_P2 * MABEN_N),
      maben.astype(bf), key_W.astype(bf), key_b.reshape(1, D),
      query_W.astype(bf), query_b.reshape(1, D))

    out = pl.pallas_call(
        _topk_body,
        out_shape=jax.ShapeDtypeStruct((bsz, TOP_K, D), f32),
    )(sim, pe)
    return out


# speedup vs baseline: 3.4923x; 1.3812x over previous
import math

import jax
import jax.numpy as jnp
from jax.experimental import pallas as pl

D = 768
DEPTH = 3
HEADS = 12
DH = 64
NUM_P2 = 32
MABEN_N = 256
TOP_K = 5
S_REAL = 453
S_X = 197
S_PAD = 456
NEG = -1e30


def _gelu(x):
    return x * 0.5 * (1.0 + jax.lax.erf(x * (1.0 / math.sqrt(2.0))))


def _ln(x, g, b, eps=1e-5):
    m = jnp.mean(x, axis=-1, keepdims=True)
    v = jnp.mean(x * x, axis=-1, keepdims=True) - m * m
    return (x - m) * jax.lax.rsqrt(v + eps) * g + b


def _attn_heads(qkv, kv_off, mask, rows):
    outs = []
    for hd in range(HEADS):
        q = qkv[0][:, hd * DH:(hd + 1) * DH]
        k = qkv[1][:, kv_off + hd * DH:kv_off + (hd + 1) * DH]
        v = qkv[1][:, kv_off + D + hd * DH:kv_off + D + (hd + 1) * DH]
        dots = jax.lax.dot_general(
            q, k, (((1,), (1,)), ((), ())),
            preferred_element_type=jnp.float32)
        e = jnp.exp(dots + mask)
        s = jnp.sum(e, axis=-1, keepdims=True)
        o = jax.lax.dot(e.astype(jnp.bfloat16), v,
                        preferred_element_type=jnp.float32)
        outs.append(o * (1.0 / s))
    return jnp.concatenate(outs, axis=1).astype(jnp.bfloat16)


def _layer(x, w, mask):
    (g1, b1, wqkv, wo, wob, g2, b2, w1, bm1, w2, bm2) = w
    h = _ln(x, g1, b1).astype(jnp.bfloat16)
    qkv = jax.lax.dot(h, wqkv,
                      preferred_element_type=jnp.float32).astype(jnp.bfloat16)
    o = _attn_heads((qkv, qkv), D, mask, S_PAD)
    x = jax.lax.dot(o, wo, preferred_element_type=jnp.float32) + wob + x
    h2 = _ln(x, g2, b2).astype(jnp.bfloat16)
    t = jax.lax.dot(h2, w1, preferred_element_type=jnp.float32) + bm1
    t = _gelu(t).astype(jnp.bfloat16)
    return jax.lax.dot(t, w2, preferred_element_type=jnp.float32) + bm2 + x


def _cls_layer(x, w, mask):
    (g1, b1, wq, wkv, wo, wob, g2, b2, w1, bm1, w2, bm2) = w
    h = _ln(x, g1, b1).astype(jnp.bfloat16)
    kv = jax.lax.dot(h, wkv,
                     preferred_element_type=jnp.float32).astype(jnp.bfloat16)
    qc = jax.lax.dot(h[0:1, :], wq,
                     preferred_element_type=jnp.float32).astype(jnp.bfloat16)
    o = _attn_heads((qc, kv), 0, mask, 1)
    xc = x[0:1, :]
    xc = jax.lax.dot(o, wo, preferred_element_type=jnp.float32) + wob + xc
    h2 = _ln(xc, g2, b2).astype(jnp.bfloat16)
    t = jax.lax.dot(h2, w1, preferred_element_type=jnp.float32) + bm1
    t = _gelu(t).astype(jnp.bfloat16)
    return jax.lax.dot(t, w2, preferred_element_type=jnp.float32) + bm2 + xc


def _fused_body(*refs):
    x_ref = refs[0]
    w01 = refs[1:23]
    wc = refs[23:35]
    cls_ref, xm_ref = refs[35], refs[36]
    x = x_ref[...]
    xm_ref[...] = jnp.mean(x[0:S_X, :], axis=0, keepdims=True)
    col = jax.lax.broadcasted_iota(jnp.int32, (1, S_PAD), 1)
    mask = jnp.where(col < S_REAL, 0.0, NEG)
    x = _layer(x, tuple(r[...] for r in w01[0:11]), mask)
    x = _layer(x, tuple(r[...] for r in w01[11:22]), mask)
    cls_ref[...] = _cls_layer(x, tuple(r[...] for r in wc), mask)


def _head_body(cls_ref, xm_ref, pow_ref, pob_ref, maben_ref, kw_ref, kb_ref,
               qw_ref, qb_ref, pe_ref):
    cls = cls_ref[...].astype(jnp.bfloat16)
    corr = jax.lax.dot(cls, pow_ref[...],
                       preferred_element_type=jnp.float32) + pob_ref[...]
    xm = xm_ref[...]
    k_ = jax.lax.dot(xm.astype(jnp.bfloat16), kw_ref[...],
                     preferred_element_type=jnp.float32) + kb_ref[...]
    kq = jax.lax.dot_general(
        k_.astype(jnp.bfloat16), qw_ref[...], (((1,), (1,)), ((), ())),
        preferred_element_type=jnp.float32)
    qbk = jnp.sum(k_ * qb_ref[...], axis=1, keepdims=True)
    inv = 1.0 / math.sqrt(D)
    for o in range(NUM_P2):
        c = corr[:, o * MABEN_N:(o + 1) * MABEN_N]
        m = jax.nn.softmax(c, axis=-1).astype(jnp.bfloat16)
        pg = jax.lax.dot(m, maben_ref[...], preferred_element_type=jnp.float32)
        s = (jnp.sum(pg * kq, axis=1, keepdims=True) + qbk) * inv
        pe_ref[:, o, :] = pg * jax.nn.sigmoid(s)


def _topk_body(xm_ref, pk_ref, pe_ref, out_ref):
    xm = xm_ref[...]
    xn = xm * jax.lax.rsqrt(
        jnp.maximum(jnp.sum(xm * xm, axis=-1, keepdims=True), 1e-12))
    pk = pk_ref[...]
    pn = pk * jax.lax.rsqrt(
        jnp.maximum(jnp.sum(pk * pk, axis=-1, keepdims=True), 1e-12))
    sim = jax.lax.dot_general(
        xn.astype(jnp.bfloat16), pn.astype(jnp.bfloat16),
        (((1,), (1,)), ((), ())), preferred_element_type=jnp.float32)
    b = sim.shape[0]
    colio = jax.lax.broadcasted_iota(jnp.int32, (b, NUM_P2), 1)
    pe = pe_ref[...]
    masked = sim
    for k in range(TOP_K):
        mx = jnp.max(masked, axis=1, keepdims=True)
        eq = masked >= mx
        mn = jnp.min(jnp.where(eq, colio, NUM_P2), axis=1, keepdims=True)
        pick = colio == mn
        w = pick.astype(jnp.float32)
        out_ref[:, k, :] = jnp.sum(w[:, :, None] * pe, axis=1)
        masked = jnp.where(pick, NEG, masked)


def _full(a):
    return pl.BlockSpec(a.shape, lambda i: (0,) * a.ndim)


def kernel(x_embed, maben, prompt_key, ln1_g, ln1_b, Wqkv, Wo, Wo_b, ln2_g,
           ln2_b, W1, b1, W2, b2, pre_out_W, pre_out_b, key_W, key_b, query_W,
           query_b):
    f32 = jnp.float32
    bf = jnp.bfloat16
    bsz = x_embed.shape[0]
    scale = 1.0 / math.sqrt(DH)

    pt = jnp.broadcast_to(maben[None], (bsz, MABEN_N, D))
    pad = jnp.zeros((bsz, S_PAD - S_REAL, D), f32)
    h = jnp.concatenate([x_embed, pt, pad], axis=1)

    def layer_weights(l):
        wqkv = jnp.concatenate([Wqkv[l][:, :D] * scale, Wqkv[l][:, D:]],
                               axis=1).astype(bf)
        return (ln1_g[l].reshape(1, D), ln1_b[l].reshape(1, D),
                wqkv, Wo[l].astype(bf), Wo_b[l].reshape(1, D),
                ln2_g[l].reshape(1, D), ln2_b[l].reshape(1, D),
                W1[l].astype(bf), b1[l].reshape(1, D),
                W2[l].astype(bf), b2[l].reshape(1, D))

    lc = DEPTH - 1
    wc = (ln1_g[lc].reshape(1, D), ln1_b[lc].reshape(1, D),
          (Wqkv[lc][:, :D] * scale).astype(bf), Wqkv[lc][:, D:].astype(bf),
          Wo[lc].astype(bf), Wo_b[lc].reshape(1, D),
          ln2_g[lc].reshape(1, D), ln2_b[lc].reshape(1, D),
          W1[lc].astype(bf), b1[lc].reshape(1, D),
          W2[lc].astype(bf), b2[lc].reshape(1, D))

    wargs = layer_weights(0) + layer_weights(1) + wc

    cls, xm = pl.pallas_call(
        _fused_body,
        grid=(bsz,),
        in_specs=[pl.BlockSpec((None, S_PAD, D), lambda i: (i, 0, 0))] +
                 [_full(w) for w in wargs],
        out_specs=(pl.BlockSpec((None, 1, D), lambda i: (i, 0, 0)),
                   pl.BlockSpec((None, 1, D), lambda i: (i, 0, 0))),
        out_shape=(jax.ShapeDtypeStruct((bsz, 1, D), f32),
                   jax.ShapeDtypeStruct((bsz, 1, D), f32)),
    )(h, *wargs)
    cls = cls.reshape(bsz, D)
    xm = xm.reshape(bsz, D)

    pe = pl.pallas_call(
        _head_body,
        out_shape=jax.ShapeDtypeStruct((bsz, NUM_P2, D), f32),
    )(cls, xm, pre_out_W.astype(bf), pre_out_b.reshape(1, NUM_P2 * MABEN_N),
      maben.astype(bf), key_W.astype(bf), key_b.reshape(1, D),
      query_W.astype(bf), query_b.reshape(1, D))

    out = pl.pallas_call(
        _topk_body,
        out_shape=jax.ShapeDtypeStruct((bsz, TOP_K, D), f32),
    )(xm, prompt_key, pe)
    return out


# speedup vs baseline: 3.7354x; 1.0696x over previous
import math

import jax
import jax.numpy as jnp
from jax.experimental import pallas as pl

D = 768
DEPTH = 3
HEADS = 12
DH = 64
NUM_P2 = 32
MABEN_N = 256
TOP_K = 5
S_REAL = 453
S_X = 197
N_PAD = 3
S_PAD = 456
NEG = -1e30


def _gelu(x):
    return x * 0.5 * (1.0 + jax.lax.erf(x * (1.0 / math.sqrt(2.0))))


def _ln(x, g, b, eps=1e-5):
    m = jnp.mean(x, axis=-1, keepdims=True)
    v = jnp.mean(x * x, axis=-1, keepdims=True) - m * m
    return (x - m) * jax.lax.rsqrt(v + eps) * g + b


def _attn_heads(q_all, kv, kv_off):
    outs = []
    for hd in range(HEADS):
        q = q_all[:, hd * DH:(hd + 1) * DH]
        k = kv[:, kv_off + hd * DH:kv_off + (hd + 1) * DH]
        v = kv[:, kv_off + D + hd * DH:kv_off + D + (hd + 1) * DH]
        dots = jax.lax.dot_general(
            q, k, (((1,), (1,)), ((), ())),
            preferred_element_type=jnp.float32)
        e = jnp.exp(dots)
        s = jnp.sum(e, axis=-1, keepdims=True) - float(N_PAD)
        o = jax.lax.dot(e.astype(jnp.bfloat16), v,
                        preferred_element_type=jnp.float32)
        outs.append(o * (1.0 / s))
    return jnp.concatenate(outs, axis=1).astype(jnp.bfloat16)


def _layer(x, w, row_ok):
    (g1, b1, wqkv, wo, wob, g2, b2, w1, bm1, w2, bm2) = w
    h = jnp.where(row_ok, _ln(x, g1, b1), 0.0).astype(jnp.bfloat16)
    qkv = jax.lax.dot(h, wqkv,
                      preferred_element_type=jnp.float32).astype(jnp.bfloat16)
    o = _attn_heads(qkv, qkv, D)
    x = jax.lax.dot(o, wo, preferred_element_type=jnp.float32) + wob + x
    h2 = _ln(x, g2, b2).astype(jnp.bfloat16)
    t = jax.lax.dot(h2, w1, preferred_element_type=jnp.float32) + bm1
    t = _gelu(t).astype(jnp.bfloat16)
    return jax.lax.dot(t, w2, preferred_element_type=jnp.float32) + bm2 + x


def _cls_layer(x, w, row_ok):
    (g1, b1, wq, wkv, wo, wob, g2, b2, w1, bm1, w2, bm2) = w
    h = jnp.where(row_ok, _ln(x, g1, b1), 0.0).astype(jnp.bfloat16)
    kv = jax.lax.dot(h, wkv,
                     preferred_element_type=jnp.float32).astype(jnp.bfloat16)
    qc = jax.lax.dot(h[0:1, :], wq,
                     preferred_element_type=jnp.float32).astype(jnp.bfloat16)
    o = _attn_heads(qc, kv, 0)
    xc = x[0:1, :]
    xc = jax.lax.dot(o, wo, preferred_element_type=jnp.float32) + wob + xc
    h2 = _ln(xc, g2, b2).astype(jnp.bfloat16)
    t = jax.lax.dot(h2, w1, preferred_element_type=jnp.float32) + bm1
    t = _gelu(t).astype(jnp.bfloat16)
    return jax.lax.dot(t, w2, preferred_element_type=jnp.float32) + bm2 + xc


def _fused_body(*refs):
    xe_ref, mab_ref = refs[0], refs[1]
    w01 = refs[2:24]
    wc = refs[24:36]
    cls_ref, xm_ref = refs[36], refs[37]
    xe = xe_ref[...]
    xm_ref[...] = jnp.mean(xe, axis=0, keepdims=True)
    x = jnp.concatenate(
        [xe, jnp.zeros((N_PAD, D), jnp.float32), mab_ref[...]], axis=0)
    row = jax.lax.broadcasted_iota(jnp.int32, (S_PAD, 1), 0)
    row_ok = (row < S_X) | (row >= S_X + N_PAD)
    x = _layer(x, tuple(r[...] for r in w01[0:11]), row_ok)
    x = _layer(x, tuple(r[...] for r in w01[11:22]), row_ok)
    cls_ref[...] = _cls_layer(x, tuple(r[...] for r in wc), row_ok)


def _head_body(cls_ref, xm_ref, pow_ref, pob_ref, maben_ref, kw_ref, kb_ref,
               qw_ref, qb_ref, pe_ref):
    cls = cls_ref[...].astype(jnp.bfloat16)
    corr = jax.lax.dot(cls, pow_ref[...],
                       preferred_element_type=jnp.float32) + pob_ref[...]
    xm = xm_ref[...]
    k_ = jax.lax.dot(xm.astype(jnp.bfloat16), kw_ref[...],
                     preferred_element_type=jnp.float32) + kb_ref[...]
    kq = jax.lax.dot_general(
        k_.astype(jnp.bfloat16), qw_ref[...], (((1,), (1,)), ((), ())),
        preferred_element_type=jnp.float32)
    qbk = jnp.sum(k_ * qb_ref[...], axis=1, keepdims=True)
    inv = 1.0 / math.sqrt(D)
    for o in range(NUM_P2):
        c = corr[:, o * MABEN_N:(o + 1) * MABEN_N]
        m = jax.nn.softmax(c, axis=-1).astype(jnp.bfloat16)
        pg = jax.lax.dot(m, maben_ref[...], preferred_element_type=jnp.float32)
        s = (jnp.sum(pg * kq, axis=1, keepdims=True) + qbk) * inv
        pe_ref[:, o, :] = pg * jax.nn.sigmoid(s)


def _topk_body(xm_ref, pk_ref, pe_ref, out_ref):
    xm = xm_ref[...]
    xn = xm * jax.lax.rsqrt(
        jnp.maximum(jnp.sum(xm * xm, axis=-1, keepdims=True), 1e-12))
    pk = pk_ref[...]
    pn = pk * jax.lax.rsqrt(
        jnp.maximum(jnp.sum(pk * pk, axis=-1, keepdims=True), 1e-12))
    sim = jax.lax.dot_general(
        xn.astype(jnp.bfloat16), pn.astype(jnp.bfloat16),
        (((1,), (1,)), ((), ())), preferred_element_type=jnp.float32)
    b = sim.shape[0]
    colio = jax.lax.broadcasted_iota(jnp.int32, (b, NUM_P2), 1)
    pe = pe_ref[...]
    masked = sim
    for k in range(TOP_K):
        mx = jnp.max(masked, axis=1, keepdims=True)
        eq = masked >= mx
        mn = jnp.min(jnp.where(eq, colio, NUM_P2), axis=1, keepdims=True)
        pick = colio == mn
        w = pick.astype(jnp.float32)
        out_ref[:, k, :] = jnp.sum(w[:, :, None] * pe, axis=1)
        masked = jnp.where(pick, NEG, masked)


def _full(a):
    return pl.BlockSpec(a.shape, lambda i: (0,) * a.ndim)


def kernel(x_embed, maben, prompt_key, ln1_g, ln1_b, Wqkv, Wo, Wo_b, ln2_g,
           ln2_b, W1, b1, W2, b2, pre_out_W, pre_out_b, key_W, key_b, query_W,
           query_b):
    f32 = jnp.float32
    bf = jnp.bfloat16
    bsz = x_embed.shape[0]
    scale = 1.0 / math.sqrt(DH)

    def layer_weights(l):
        wqkv = jnp.concatenate([Wqkv[l][:, :D] * scale, Wqkv[l][:, D:]],
                               axis=1).astype(bf)
        return (ln1_g[l].reshape(1, D), ln1_b[l].reshape(1, D),
                wqkv, Wo[l].astype(bf), Wo_b[l].reshape(1, D),
                ln2_g[l].reshape(1, D), ln2_b[l].reshape(1, D),
                W1[l].astype(bf), b1[l].reshape(1, D),
                W2[l].astype(bf), b2[l].reshape(1, D))

    lc = DEPTH - 1
    wc = (ln1_g[lc].reshape(1, D), ln1_b[lc].reshape(1, D),
          (Wqkv[lc][:, :D] * scale).astype(bf), Wqkv[lc][:, D:].astype(bf),
          Wo[lc].astype(bf), Wo_b[lc].reshape(1, D),
          ln2_g[lc].reshape(1, D), ln2_b[lc].reshape(1, D),
          W1[lc].astype(bf), b1[lc].reshape(1, D),
          W2[lc].astype(bf), b2[lc].reshape(1, D))

    wargs = layer_weights(0) + layer_weights(1) + wc

    cls, xm = pl.pallas_call(
        _fused_body,
        grid=(bsz,),
        in_specs=[pl.BlockSpec((None, S_X, D), lambda i: (i, 0, 0)),
                  _full(maben)] +
                 [_full(w) for w in wargs],
        out_specs=(pl.BlockSpec((None, 1, D), lambda i: (i, 0, 0)),
                   pl.BlockSpec((None, 1, D), lambda i: (i, 0, 0))),
        out_shape=(jax.ShapeDtypeStruct((bsz, 1, D), f32),
                   jax.ShapeDtypeStruct((bsz, 1, D), f32)),
    )(x_embed, maben, *wargs)
    cls = cls.reshape(bsz, D)
    xm = xm.reshape(bsz, D)

    pe = pl.pallas_call(
        _head_body,
        out_shape=jax.ShapeDtypeStruct((bsz, NUM_P2, D), f32),
    )(cls, xm, pre_out_W.astype(bf), pre_out_b.reshape(1, NUM_P2 * MABEN_N),
      maben.astype(bf), key_W.astype(bf), key_b.reshape(1, D),
      query_W.astype(bf), query_b.reshape(1, D))

    out = pl.pallas_call(
        _topk_body,
        out_shape=jax.ShapeDtypeStruct((bsz, TOP_K, D), f32),
    )(xm, prompt_key, pe)
    return out
